# baseline (speedup 1.0000x reference)
import functools

import jax
import jax.numpy as jnp
from jax import lax
from jax.experimental import pallas as pl
from jax.experimental.pallas import tpu as pltpu
from jax.experimental.pallas import tpu_sc as plsc

N_NODES = 10000
N_PAD = 10240
DUMMY = N_NODES
N_EDGES = 320000
NSUB = 16
CHUNK = 128
NBUF = 4
TOT_CH = 160
E_PAD = NSUB * TOT_CH * CHUNK
ZROWS = N_PAD // NSUB
MBLK = 2048
GRID = N_PAD // MBLK

NCH1_DEG = 80
NCH1_64 = 80
NCH1_32 = 80
NCH1_16 = 80

_mesh = plsc.VectorSubcoreMesh(core_axis_name="c", subcore_axis_name="s")
_sc_params = pltpu.CompilerParams(use_tc_tiling_on_sc=False)


@functools.partial(
    pl.kernel,
    out_type=(jax.ShapeDtypeStruct((N_PAD,), jnp.float32),
              jax.ShapeDtypeStruct((N_PAD,), jnp.float32)),
    mesh=_mesh,
    scratch_types=[
        pltpu.VMEM((max(NCH1_DEG, TOT_CH - NCH1_DEG), CHUNK), jnp.int32),
        pltpu.VMEM((CHUNK,), jnp.float32),
        pltpu.VMEM((ZROWS,), jnp.float32),
        pltpu.VMEM_SHARED((N_PAD,), jnp.float32),
    ] + [pltpu.SemaphoreType.DMA] * NBUF,
    compiler_params=_sc_params,
)
def _deg_kernel(dst_hbm, out0_hbm, out1_hbm, idx_v, ones_v, zer_v, acc,
                *sems):
    c = lax.axis_index("c")
    s = lax.axis_index("s")

    def fill_ones(i, carry):
        ones_v[pl.ds(i * 16, 16)] = jnp.full((16,), 1.0, jnp.float32)
        return carry

    lax.fori_loop(0, CHUNK // 16, fill_ones, 0)

    def fill_zeros(i, carry):
        zer_v[pl.ds(i * 16, 16)] = jnp.zeros((16,), jnp.float32)
        return carry

    lax.fori_loop(0, ZROWS // 16, fill_zeros, 0)

    pltpu.sync_copy(zer_v, acc.at[pl.ds(s * ZROWS, ZROWS)])

    def body_for(coff, nch):
        pltpu.sync_copy(dst_hbm.at[s, pl.ds(coff, nch)],
                        idx_v.at[pl.ds(0, nch)])
        plsc.subcore_barrier()

        def start(j, b):
            pltpu.async_copy(ones_v, acc.at[idx_v.at[j]], sems[b], add=True)

        def wait(j, b):
            pltpu.make_async_copy(ones_v, acc.at[idx_v.at[j]],
                                  sems[b]).wait()

        for b in range(NBUF):
            start(b, b)

        def body(i, carry):
            for b in range(NBUF):
                j = i * NBUF + b
                wait(j, b)
                start(j + NBUF, b)
            return carry

        lax.fori_loop(0, nch // NBUF - 1, body, 0)
        for b in range(NBUF):
            wait(nch - NBUF + b, b)

    @pl.when(c == 0)
    def _():
        body_for(0, TOT_CH - NCH1_DEG)

    @pl.when(c == 1)
    def _():
        body_for(TOT_CH - NCH1_DEG, NCH1_DEG)

    plsc.subcore_barrier()

    @pl.when(c == 0)
    def _():
        pltpu.sync_copy(acc.at[pl.ds(s * ZROWS, ZROWS)],
                        out0_hbm.at[pl.ds(s * ZROWS, ZROWS)])

    @pl.when(c == 1)
    def _():
        pltpu.sync_copy(acc.at[pl.ds(s * ZROWS, ZROWS)],
                        out1_hbm.at[pl.ds(s * ZROWS, ZROWS)])


def _make_agg(h, nch1):
    nch0 = TOT_CH - nch1

    @functools.partial(
        pl.kernel,
        out_type=jax.ShapeDtypeStruct((2, N_PAD, h), jnp.float32),
        mesh=_mesh,
        scratch_types=[
            pltpu.VMEM((max(nch0, nch1), CHUNK), jnp.int32),
            pltpu.VMEM((max(nch0, nch1), CHUNK), jnp.int32),
            pltpu.VMEM((NBUF, CHUNK, h), jnp.float32),
            pltpu.VMEM((CHUNK, h), jnp.float32),
            pltpu.VMEM_SHARED((N_PAD, h), jnp.float32),
        ] + [pltpu.SemaphoreType.DMA] * (2 * NBUF),
        compiler_params=_sc_params,
    )
    def _agg(g_hbm, src_hbm, dst_hbm, out_hbm, si_v, di_v, rows_v,
             zb_v, acc, *sems):
        gsem = sems[:NBUF]
        ssem = sems[NBUF:]
        c = lax.axis_index("c")
        s = lax.axis_index("s")

        def fill_zeros(i, carry):
            for k in range(h // 16):
                zb_v[i, pl.ds(k * 16, 16)] = jnp.zeros((16,), jnp.float32)
            return carry

        lax.fori_loop(0, CHUNK, fill_zeros, 0)
        for q in range(ZROWS // CHUNK):
            pltpu.sync_copy(zb_v, acc.at[pl.ds(s * ZROWS + q * CHUNK, CHUNK)])

        def gather_start(j, b):
            pltpu.async_copy(g_hbm.at[si_v.at[j]], rows_v.at[b], gsem[b])

        def gather_wait(j, b):
            pltpu.make_async_copy(g_hbm.at[si_v.at[j]], rows_v.at[b],
                                  gsem[b]).wait()

        def scatter_start(j, b):
            pltpu.async_copy(rows_v.at[b], acc.at[di_v.at[j]], ssem[b],
                             add=True)

        def scatter_wait(j, b):
            pltpu.make_async_copy(rows_v.at[b], acc.at[di_v.at[j]],
                                  ssem[b]).wait()

        def run_ring(coff, nch):
            pltpu.sync_copy(src_hbm.at[s, pl.ds(coff, nch)],
                            si_v.at[pl.ds(0, nch)])
            pltpu.sync_copy(dst_hbm.at[s, pl.ds(coff, nch)],
                            di_v.at[pl.ds(0, nch)])
            plsc.subcore_barrier()
            for b in range(NBUF):
                gather_start(b, b)

            def body(i, carry):
                for b in range(NBUF):
                    j = i * NBUF + b
                    gather_wait(j, b)
                    scatter_start(j, b)
                    scatter_wait(j, b)
                    gather_start(j + NBUF, b)
                return carry

            lax.fori_loop(0, nch // NBUF - 1, body, 0)
            for b in range(NBUF):
                j = nch - NBUF + b
                gather_wait(j, b)
                scatter_start(j, b)
                scatter_wait(j, b)

        @pl.when(c == 0)
        def _():
            run_ring(0, nch0)

        @pl.when(c == 1)
        def _():
            run_ring(nch0, nch1)

        plsc.subcore_barrier()
        pltpu.sync_copy(acc.at[pl.ds(s * ZROWS, ZROWS)],
                        out_hbm.at[c].at[pl.ds(s * ZROWS, ZROWS)])

    return _agg


_agg64 = _make_agg(64, NCH1_64)
_agg32 = _make_agg(32, NCH1_32)
_agg16 = _make_agg(16, NCH1_16)


def _dinv(d0_ref, d1_ref, rows):
    d = d0_ref[...] + d1_ref[...] + 1.0
    return lax.rsqrt(jnp.maximum(d, 1.0)).reshape(rows, 1)


def _t_raw(x_ref, w_ref, o_ref):
    o_ref[...] = jnp.dot(x_ref[...], w_ref[...],
                         preferred_element_type=jnp.float32)


def _t_scale(d0_ref, d1_ref, r_ref, o_ref):
    o_ref[...] = _dinv(d0_ref, d1_ref, MBLK) * r_ref[...]


def _t_mid(d0_ref, d1_ref, p_ref, g_ref, b_ref, w_ref, o_ref):
    dinv = _dinv(d0_ref, d1_ref, MBLK)
    agg = p_ref[0] + p_ref[1] + g_ref[...]
    hid = jnp.maximum(dinv * agg + b_ref[...], 0.0)
    o_ref[...] = dinv * jnp.dot(hid, w_ref[...],
                                preferred_element_type=jnp.float32)


def _t_last(d0_ref, d1_ref, p_ref, g_ref, b_ref, w_ref, bc_ref, o_ref):
    dinv = _dinv(d0_ref, d1_ref, MBLK)
    agg = p_ref[0] + p_ref[1] + g_ref[...]
    hid = jnp.maximum(dinv * agg + b_ref[...], 0.0)
    o_ref[...] = jnp.dot(hid, w_ref[...],
                         preferred_element_type=jnp.float32) + bc_ref[...]


def _d_spec():
    return pl.BlockSpec((MBLK,), lambda m: (m,))


def _row_spec(h):
    return pl.BlockSpec((MBLK, h), lambda m: (m, 0))


def _full_spec(shape):
    return pl.BlockSpec(shape, lambda m: tuple(0 for _ in shape))


def _call_raw(xp, w1):
    f_in, h = w1.shape
    return pl.pallas_call(
        _t_raw,
        grid=(GRID,),
        in_specs=[_row_spec(f_in), _full_spec((f_in, h))],
        out_specs=_row_spec(h),
        out_shape=jax.ShapeDtypeStruct((N_PAD, h), jnp.float32),
    )(xp, w1)


def _call_scale(d0, d1, raw):
    h = raw.shape[1]
    return pl.pallas_call(
        _t_scale,
        grid=(GRID,),
        in_specs=[_d_spec(), _d_spec(), _row_spec(h)],
        out_specs=_row_spec(h),
        out_shape=jax.ShapeDtypeStruct((N_PAD, h), jnp.float32),
    )(d0, d1, raw)


def _p_spec(h):
    return pl.BlockSpec((2, MBLK, h), lambda m: (0, m, 0))


def _call_mid(d0, d1, p, g, b, w):
    hp, hn = w.shape
    return pl.pallas_call(
        _t_mid,
        grid=(GRID,),
        in_specs=[_d_spec(), _d_spec(), _p_spec(hp),
                  _row_spec(hp), _full_spec((1, hp)), _full_spec((hp, hn))],
        out_specs=_row_spec(hn),
        out_shape=jax.ShapeDtypeStruct((N_PAD, hn), jnp.float32),
    )(d0, d1, p, g, b.reshape(1, hp), w)


def _call_last(d0, d1, p, g, b, wc, bc):
    hp, nc = wc.shape
    return pl.pallas_call(
        _t_last,
        grid=(GRID,),
        in_specs=[_d_spec(), _d_spec(), _p_spec(hp),
                  _row_spec(hp), _full_spec((1, hp)), _full_spec((hp, nc)),
                  _full_spec((1, nc))],
        out_specs=pl.BlockSpec((MBLK, nc), lambda m: (m, 0)),
        out_shape=jax.ShapeDtypeStruct((N_NODES, nc), jnp.float32),
    )(d0, d1, p, g, b.reshape(1, hp), wc, bc.reshape(1, nc))


def kernel(x, edge_index, W1, b1, W2, b2, W3, b3, Wc, bc):
    n = x.shape[0]
    tail = DUMMY + jnp.arange(E_PAD - N_EDGES, dtype=jnp.int32) % (N_PAD - N_NODES)
    src = jnp.concatenate([edge_index[0], tail]).reshape(NSUB, TOT_CH, CHUNK)
    dst = jnp.concatenate([edge_index[1], tail]).reshape(NSUB, TOT_CH, CHUNK)
    xp = jnp.pad(x, ((0, N_PAD - n), (0, 0)))

    d0, d1 = _deg_kernel(dst)
    raw1 = _call_raw(xp, W1)

    g1 = _call_scale(d0, d1, raw1)
    p1 = _agg64(g1, src, dst)
    g2 = _call_mid(d0, d1, p1, g1, b1, W2)
    p2 = _agg32(g2, src, dst)
    g3 = _call_mid(d0, d1, p2, g2, b2, W3)
    p3 = _agg16(g3, src, dst)
    return _call_last(d0, d1, p3, g3, b3, Wc, bc)


# speedup vs baseline: 1.0550x; 1.0550x over previous
import functools

import jax
import jax.numpy as jnp
from jax import lax
from jax.experimental import pallas as pl
from jax.experimental.pallas import tpu as pltpu
from jax.experimental.pallas import tpu_sc as plsc

N_NODES = 10000
N_PAD = 10240
DUMMY = N_NODES
N_EDGES = 320000
NSUB = 16
CHUNK = 128
NBUF = 4
TOT_CH = 160
E_PAD = NSUB * TOT_CH * CHUNK
ZROWS = N_PAD // NSUB
MBLK = 2048
GRID = N_PAD // MBLK

NCH1_DEG = 80
NCH1_64 = 80
NCH1_32 = 80
NCH1_16 = 80

_mesh = plsc.VectorSubcoreMesh(core_axis_name="c", subcore_axis_name="s")
_sc_params = pltpu.CompilerParams(use_tc_tiling_on_sc=False)


@functools.partial(
    pl.kernel,
    out_type=(jax.ShapeDtypeStruct((N_PAD,), jnp.float32),
              jax.ShapeDtypeStruct((N_PAD,), jnp.float32)),
    mesh=_mesh,
    scratch_types=[
        pltpu.VMEM((max(NCH1_DEG, TOT_CH - NCH1_DEG), CHUNK), jnp.int32),
        pltpu.VMEM((CHUNK,), jnp.float32),
        pltpu.VMEM((ZROWS,), jnp.float32),
        pltpu.VMEM_SHARED((N_PAD,), jnp.float32),
    ] + [pltpu.SemaphoreType.DMA] * NBUF,
    compiler_params=_sc_params,
)
def _deg_kernel(dst_hbm, out0_hbm, out1_hbm, idx_v, ones_v, zer_v, acc,
                *sems):
    c = lax.axis_index("c")
    s = lax.axis_index("s")

    def fill_ones(i, carry):
        ones_v[pl.ds(i * 16, 16)] = jnp.full((16,), 1.0, jnp.float32)
        return carry

    lax.fori_loop(0, CHUNK // 16, fill_ones, 0)

    def fill_zeros(i, carry):
        zer_v[pl.ds(i * 16, 16)] = jnp.zeros((16,), jnp.float32)
        return carry

    lax.fori_loop(0, ZROWS // 16, fill_zeros, 0)

    pltpu.sync_copy(zer_v, acc.at[pl.ds(s * ZROWS, ZROWS)])

    def body_for(coff, nch):
        pltpu.sync_copy(dst_hbm.at[s, pl.ds(coff, nch)],
                        idx_v.at[pl.ds(0, nch)])
        plsc.subcore_barrier()

        def start(j, b):
            pltpu.async_copy(ones_v, acc.at[idx_v.at[j]], sems[b], add=True)

        def wait(j, b):
            pltpu.make_async_copy(ones_v, acc.at[idx_v.at[j]],
                                  sems[b]).wait()

        for b in range(NBUF):
            start(b, b)

        def body(i, carry):
            for b in range(NBUF):
                j = i * NBUF + b
                wait(j, b)
                start(j + NBUF, b)
            return carry

        lax.fori_loop(0, nch // NBUF - 1, body, 0)
        for b in range(NBUF):
            wait(nch - NBUF + b, b)

    @pl.when(c == 0)
    def _():
        body_for(0, TOT_CH - NCH1_DEG)

    @pl.when(c == 1)
    def _():
        body_for(TOT_CH - NCH1_DEG, NCH1_DEG)

    plsc.subcore_barrier()

    @pl.when(c == 0)
    def _():
        pltpu.sync_copy(acc.at[pl.ds(s * ZROWS, ZROWS)],
                        out0_hbm.at[pl.ds(s * ZROWS, ZROWS)])

    @pl.when(c == 1)
    def _():
        pltpu.sync_copy(acc.at[pl.ds(s * ZROWS, ZROWS)],
                        out1_hbm.at[pl.ds(s * ZROWS, ZROWS)])


def _make_agg(h, nch1):
    nch0 = TOT_CH - nch1

    @functools.partial(
        pl.kernel,
        out_type=jax.ShapeDtypeStruct((2, N_PAD, h), jnp.float32),
        mesh=_mesh,
        scratch_types=[
            pltpu.VMEM((max(nch0, nch1), CHUNK), jnp.int32),
            pltpu.VMEM((max(nch0, nch1), CHUNK), jnp.int32),
            pltpu.VMEM((NBUF, CHUNK, h), jnp.float32),
            pltpu.VMEM((CHUNK, h), jnp.float32),
            pltpu.VMEM_SHARED((N_PAD, h), jnp.float32),
        ] + [pltpu.SemaphoreType.DMA] * (2 * NBUF),
        compiler_params=_sc_params,
    )
    def _agg(g_hbm, src_hbm, dst_hbm, out_hbm, si_v, di_v, rows_v,
             zb_v, acc, *sems):
        gsem = sems[:NBUF]
        ssem = sems[NBUF:]
        c = lax.axis_index("c")
        s = lax.axis_index("s")

        def fill_zeros(i, carry):
            for k in range(h // 16):
                zb_v[i, pl.ds(k * 16, 16)] = jnp.zeros((16,), jnp.float32)
            return carry

        lax.fori_loop(0, CHUNK, fill_zeros, 0)
        for q in range(ZROWS // CHUNK):
            pltpu.sync_copy(zb_v, acc.at[pl.ds(s * ZROWS + q * CHUNK, CHUNK)])

        def gather_start(j, b):
            pltpu.async_copy(g_hbm.at[si_v.at[j]], rows_v.at[b], gsem[b])

        def gather_wait(j, b):
            pltpu.make_async_copy(g_hbm.at[si_v.at[j]], rows_v.at[b],
                                  gsem[b]).wait()

        def scatter_start(j, b):
            pltpu.async_copy(rows_v.at[b], acc.at[di_v.at[j]], ssem[b],
                             add=True)

        def scatter_wait(j, b):
            pltpu.make_async_copy(rows_v.at[b], acc.at[di_v.at[j]],
                                  ssem[b]).wait()

        def run_ring(coff, nch):
            pltpu.sync_copy(src_hbm.at[s, pl.ds(coff, nch)],
                            si_v.at[pl.ds(0, nch)])
            pltpu.sync_copy(dst_hbm.at[s, pl.ds(coff, nch)],
                            di_v.at[pl.ds(0, nch)])
            plsc.subcore_barrier()
            for b in range(NBUF):
                gather_start(b, b)

            def body(i, carry):
                for b in range(NBUF):
                    j = i * NBUF + b
                    gather_wait(j, b)
                    scatter_start(j, b)
                    scatter_wait(j, b)
                    gather_start(j + NBUF, b)
                return carry

            lax.fori_loop(0, nch // NBUF - 1, body, 0)
            for b in range(NBUF):
                j = nch - NBUF + b
                gather_wait(j, b)
                scatter_start(j, b)
                scatter_wait(j, b)

        @pl.when(c == 0)
        def _():
            run_ring(0, nch0)

        @pl.when(c == 1)
        def _():
            run_ring(nch0, nch1)

        plsc.subcore_barrier()
        pltpu.sync_copy(acc.at[pl.ds(s * ZROWS, ZROWS)],
                        out_hbm.at[c].at[pl.ds(s * ZROWS, ZROWS)])

    return _agg


_agg64 = _make_agg(64, NCH1_64)
_agg32 = _make_agg(32, NCH1_32)
_agg16 = _make_agg(16, NCH1_16)


def _dinv(d0_ref, d1_ref, rows):
    d = d0_ref[...] + d1_ref[...] + 1.0
    return lax.rsqrt(jnp.maximum(d, 1.0)).reshape(rows, 1)


def _t_first(d0_ref, d1_ref, x_ref, w_ref, o_ref):
    o_ref[...] = _dinv(d0_ref, d1_ref, MBLK) * jnp.dot(
        x_ref[...], w_ref[...], preferred_element_type=jnp.float32)


def _t_mid(d0_ref, d1_ref, p_ref, g_ref, b_ref, w_ref, o_ref):
    dinv = _dinv(d0_ref, d1_ref, MBLK)
    agg = p_ref[0] + p_ref[1] + g_ref[...]
    hid = jnp.maximum(dinv * agg + b_ref[...], 0.0)
    o_ref[...] = dinv * jnp.dot(hid, w_ref[...],
                                preferred_element_type=jnp.float32)


def _t_last(d0_ref, d1_ref, p_ref, g_ref, b_ref, w_ref, bc_ref, o_ref):
    dinv = _dinv(d0_ref, d1_ref, MBLK)
    agg = p_ref[0] + p_ref[1] + g_ref[...]
    hid = jnp.maximum(dinv * agg + b_ref[...], 0.0)
    o_ref[...] = jnp.dot(hid, w_ref[...],
                         preferred_element_type=jnp.float32) + bc_ref[...]


def _d_spec():
    return pl.BlockSpec((MBLK,), lambda m: (m,))


def _row_spec(h):
    return pl.BlockSpec((MBLK, h), lambda m: (m, 0))


def _full_spec(shape):
    return pl.BlockSpec(shape, lambda m: tuple(0 for _ in shape))


def _call_first(d0, d1, xp, w1):
    f_in, h = w1.shape
    return pl.pallas_call(
        _t_first,
        grid=(GRID,),
        in_specs=[_d_spec(), _d_spec(), _row_spec(f_in),
                  _full_spec((f_in, h))],
        out_specs=_row_spec(h),
        out_shape=jax.ShapeDtypeStruct((N_PAD, h), jnp.float32),
    )(d0, d1, xp, w1)


def _p_spec(h):
    return pl.BlockSpec((2, MBLK, h), lambda m: (0, m, 0))


def _call_mid(d0, d1, p, g, b, w):
    hp, hn = w.shape
    return pl.pallas_call(
        _t_mid,
        grid=(GRID,),
        in_specs=[_d_spec(), _d_spec(), _p_spec(hp),
                  _row_spec(hp), _full_spec((1, hp)), _full_spec((hp, hn))],
        out_specs=_row_spec(hn),
        out_shape=jax.ShapeDtypeStruct((N_PAD, hn), jnp.float32),
    )(d0, d1, p, g, b.reshape(1, hp), w)


def _call_last(d0, d1, p, g, b, wc, bc):
    hp, nc = wc.shape
    return pl.pallas_call(
        _t_last,
        grid=(GRID,),
        in_specs=[_d_spec(), _d_spec(), _p_spec(hp),
                  _row_spec(hp), _full_spec((1, hp)), _full_spec((hp, nc)),
                  _full_spec((1, nc))],
        out_specs=pl.BlockSpec((MBLK, nc), lambda m: (m, 0)),
        out_shape=jax.ShapeDtypeStruct((N_NODES, nc), jnp.float32),
    )(d0, d1, p, g, b.reshape(1, hp), wc, bc.reshape(1, nc))


def kernel(x, edge_index, W1, b1, W2, b2, W3, b3, Wc, bc):
    n = x.shape[0]
    tail = DUMMY + jnp.arange(E_PAD - N_EDGES, dtype=jnp.int32) % (N_PAD - N_NODES)
    ei = jnp.concatenate(
        [edge_index, jnp.broadcast_to(tail, (2, E_PAD - N_EDGES))], axis=1)
    src = ei[0].reshape(NSUB, TOT_CH, CHUNK)
    dst = ei[1].reshape(NSUB, TOT_CH, CHUNK)
    xp = jnp.pad(x, ((0, N_PAD - n), (0, 0)))

    d0, d1 = _deg_kernel(dst)
    g1 = _call_first(d0, d1, xp, W1)
    p1 = _agg64(g1, src, dst)
    g2 = _call_mid(d0, d1, p1, g1, b1, W2)
    p2 = _agg32(g2, src, dst)
    g3 = _call_mid(d0, d1, p2, g2, b2, W3)
    p3 = _agg16(g3, src, dst)
    return _call_last(d0, d1, p3, g3, b3, Wc, bc)
